# baseline (speedup 1.0000x reference)
import functools

import jax
import jax.numpy as jnp
from jax import lax
from jax.experimental import pallas as pl
from jax.experimental.pallas import tpu as pltpu
from jax.experimental.pallas import tpu_sc as plsc

B = 4096
L = 200
EMB = 128

NW = 32
N = B * L
PER_W = N // NW
CHUNK = 128
NCHUNK = PER_W // CHUNK
NBUF = 4

_mesh = plsc.VectorSubcoreMesh(core_axis_name="c", subcore_axis_name="s")


@functools.partial(
    pl.kernel,
    out_type=jax.ShapeDtypeStruct((N, EMB), jnp.float32),
    mesh=_mesh,
    scratch_types=[
        pltpu.VMEM((NCHUNK, CHUNK), jnp.int32),
        [pltpu.VMEM((CHUNK, EMB), jnp.float32)] * NBUF,
        [pltpu.SemaphoreType.DMA] * NBUF,
        [pltpu.SemaphoreType.DMA] * NBUF,
    ],
)
def _gather_kernel(idx_hbm, table_hbm, out_hbm, idx_v, rows, gsems, wsems):
    wid = lax.axis_index("s") * 2 + lax.axis_index("c")
    base = wid * PER_W

    pltpu.sync_copy(idx_hbm.at[pl.ds(wid * NCHUNK, NCHUNK)], idx_v)

    def issue_gather(g, k):
        pltpu.async_copy(table_hbm.at[idx_v.at[g]], rows[k], gsems[k])

    def wait_gather(k):
        pltpu.make_async_copy(table_hbm.at[pl.ds(0, CHUNK)], rows[k], gsems[k]).wait()

    def issue_write(g, k):
        pltpu.async_copy(rows[k], out_hbm.at[pl.ds(base + g * CHUNK, CHUNK)], wsems[k])

    def wait_write(k):
        pltpu.make_async_copy(rows[k], out_hbm.at[pl.ds(base, CHUNK)], wsems[k]).wait()

    for k in range(NBUF):
        issue_gather(k, k)

    @pl.loop(0, NCHUNK, step=NBUF)
    def _body(g):
        for k in range(NBUF):
            wait_gather(k)
            issue_write(g + k, k)
        for k in range(NBUF):
            @pl.when(g + NBUF + k < NCHUNK)
            def _():
                wait_write(k)
                issue_gather(g + NBUF + k, k)

    for k in range(NBUF):
        wait_write(k)


def kernel(features, table):
    idx = features.reshape(NW * NCHUNK, CHUNK)
    out = _gather_kernel(idx, table)
    return out.reshape(B, L, EMB)
